# Initial kernel scaffold; baseline (speedup 1.0000x reference)
import jax
import jax.numpy as jnp
from jax.experimental import pallas as pl


def kernel(x, W1, b1, W2, b2, W_ih, W_hh, b_ih, b_hh, Wo, bo, edge_index):
    raise NotImplementedError("write your pallas kernel here")


# speedup vs baseline: 9.1190x; 9.1190x over previous
import functools

import jax
import jax.numpy as jnp
from jax import lax
from jax.experimental import pallas as pl
from jax.experimental.pallas import tpu as pltpu
from jax.experimental.pallas import tpu_sc as plsc

N = 50000
E = 800000
H = 64
F_IN = 5
K = 7
C = 5

NP = 50176
EP = 802816
NB = 512
GRID_N = NP // NB
TILES = 16
SCS = 2
BATCH = 128
NB1 = EP // (SCS * TILES * BATCH)
NB2 = EP // (TILES * BATCH)
STRIPE = NP // TILES
FC = 32
NCH1 = 2
NCH = (K * H) // FC
F1 = NCH1 * FC
F32 = jnp.float32


def _sc_mesh():
    return plsc.VectorSubcoreMesh(core_axis_name="c", subcore_axis_name="s")


@functools.cache
def _get_deg_kernel():
    return functools.partial(
        pl.kernel,
        out_type=jax.ShapeDtypeStruct((SCS, NP, 16), F32),
        mesh=_sc_mesh(),
        compiler_params=pltpu.CompilerParams(use_tc_tiling_on_sc=False),
        scratch_types=[
            pltpu.VMEM((BATCH,), jnp.int32),
            pltpu.VMEM((BATCH, 16), F32),
            pltpu.VMEM_SHARED((NP, 16), F32),
        ],
    )(_deg_body)


def _deg_body(dst3, ones16, zeros16, out, idx_v, ones_v, acc):
    cid = lax.axis_index("c")
    sid = lax.axis_index("s")
    w = cid * TILES + sid
    base = sid * STRIPE
    pltpu.sync_copy(ones16, ones_v)
    pltpu.sync_copy(zeros16, acc.at[pl.ds(base, STRIPE)])
    plsc.subcore_barrier()

    def body(b, carry):
        pltpu.sync_copy(dst3.at[w, b], idx_v)
        pltpu.sync_copy(ones_v, acc.at[idx_v], add=True)
        return carry

    lax.fori_loop(0, NB1, body, 0)
    plsc.subcore_barrier()
    pltpu.sync_copy(acc.at[pl.ds(base, STRIPE)], out.at[cid, pl.ds(base, STRIPE)])


@functools.cache
def _get_scat_kernel(nch):
    return functools.partial(
        pl.kernel,
        out_type=jax.ShapeDtypeStruct((nch * NP, FC), F32),
        mesh=_sc_mesh(),
        compiler_params=pltpu.CompilerParams(use_tc_tiling_on_sc=False),
        scratch_types=[
            pltpu.VMEM((BATCH,), jnp.int32),
            pltpu.VMEM((BATCH,), jnp.int32),
            pltpu.VMEM((BATCH, FC), F32),
            pltpu.VMEM_SHARED((NP, FC), F32),
            pltpu.SemaphoreType.DMA,
        ],
    )(functools.partial(_scat_body, nch))


def _scat_body(nch, table, src3, dst3, zerosfc, out, sidx, didx, rows, acc, sem):
    cid = lax.axis_index("c")
    sid = lax.axis_index("s")
    base = sid * STRIPE
    pltpu.sync_copy(zerosfc, acc.at[pl.ds(base, STRIPE)])

    def chunk_body(cl, carry):
        ch = cid * (nch // SCS) + cl
        off = ch * NP
        plsc.subcore_barrier()

        def body(b, c2):
            pltpu.sync_copy(src3.at[sid, b], sidx)
            pltpu.sync_copy(dst3.at[sid, b], didx)
            for q in range(BATCH // 16):
                sidx[pl.ds(q * 16, 16)] = sidx[pl.ds(q * 16, 16)] + off
            pltpu.async_copy(table.at[sidx], rows, sem).wait()
            pltpu.sync_copy(rows, acc.at[didx], add=True)
            return c2

        lax.fori_loop(0, NB2, body, 0)
        plsc.subcore_barrier()
        pltpu.sync_copy(acc.at[pl.ds(base, STRIPE)],
                        out.at[pl.ds(off + base, STRIPE)])
        pltpu.sync_copy(zerosfc, acc.at[pl.ds(base, STRIPE)])
        return carry

    lax.fori_loop(0, nch // SCS, chunk_body, 0)


def _prep_body(degp, x2, u1, dinv):
    dd = degp[...]
    d = dd[0, :, 0] + dd[1, :, 0] + 1.0
    di = 1.0 / jnp.sqrt(d)
    dinv[...] = di[:, None]
    u = x2[...] * di[:, None]
    u1[0] = u[:, :FC]
    u1[1] = u[:, FC:]


_prep = pl.pallas_call(
    _prep_body,
    grid=(GRID_N,),
    in_specs=[
        pl.BlockSpec((SCS, NB, 16), lambda i: (0, i, 0)),
        pl.BlockSpec((NB, F1), lambda i: (i, 0)),
    ],
    out_specs=[
        pl.BlockSpec((NCH1, NB, FC), lambda i: (0, i, 0)),
        pl.BlockSpec((NB, 1), lambda i: (i, 0)),
    ],
    out_shape=[
        jax.ShapeDtypeStruct((NCH1, NP, FC), F32),
        jax.ShapeDtypeStruct((NP, 1), F32),
    ],
)


def _mid_body(y1, u1, dinv, w1, b1, v):
    di = dinv[...]
    ys = jnp.concatenate([y1[0] + u1[0], y1[1] + u1[1]], axis=1) * di
    for k in range(K):
        xs = jnp.dot(ys[:, 5 * k:5 * k + 5], w1[...],
                     preferred_element_type=F32,
                     precision=lax.Precision.HIGHEST)
        xs = jnp.maximum(xs + b1[...], 0.0)
        vk = xs * di
        v[2 * k] = vk[:, :FC]
        v[2 * k + 1] = vk[:, FC:]


_mid = pl.pallas_call(
    _mid_body,
    grid=(GRID_N,),
    in_specs=[
        pl.BlockSpec((NCH1, NB, FC), lambda i: (0, i, 0)),
        pl.BlockSpec((NCH1, NB, FC), lambda i: (0, i, 0)),
        pl.BlockSpec((NB, 1), lambda i: (i, 0)),
        pl.BlockSpec((F_IN, H), lambda i: (0, 0)),
        pl.BlockSpec((1, H), lambda i: (0, 0)),
    ],
    out_specs=pl.BlockSpec((NCH, NB, FC), lambda i: (0, i, 0)),
    out_shape=jax.ShapeDtypeStruct((NCH, NP, FC), F32),
)


def _gru_body(z, vv, dinv, w2, b2, wiht, whht, bih, bhh, wo, bo, out):
    di = dinv[...]
    h = jnp.zeros((NB, H), F32)
    for k in range(K):
        za = (z[2 * k] + vv[2 * k]) * di
        zb = (z[2 * k + 1] + vv[2 * k + 1]) * di
        z2 = jnp.concatenate([za, zb], axis=1)
        xs2 = jnp.dot(z2, w2[...], preferred_element_type=F32,
                      precision=lax.Precision.HIGHEST)
        xs2 = jnp.maximum(xs2 + b2[...], 0.0)
        gi = jnp.dot(xs2, wiht[...], preferred_element_type=F32,
                     precision=lax.Precision.HIGHEST) + bih[...]
        gh = jnp.dot(h, whht[...], preferred_element_type=F32,
                     precision=lax.Precision.HIGHEST) + bhh[...]
        r = jax.nn.sigmoid(gi[:, :H] + gh[:, :H])
        zz = jax.nn.sigmoid(gi[:, H:2 * H] + gh[:, H:2 * H])
        n = jnp.tanh(gi[:, 2 * H:] + r * gh[:, 2 * H:])
        h = (1.0 - zz) * n + zz * h
    logits = jnp.dot(h, wo[...], preferred_element_type=F32,
                     precision=lax.Precision.HIGHEST) + bo[...]
    m = jnp.max(logits, axis=1, keepdims=True)
    p = jnp.exp(logits - m)
    out[...] = p / jnp.sum(p, axis=1, keepdims=True)


_gru = pl.pallas_call(
    _gru_body,
    grid=(GRID_N,),
    in_specs=[
        pl.BlockSpec((NCH, NB, FC), lambda i: (0, i, 0)),
        pl.BlockSpec((NCH, NB, FC), lambda i: (0, i, 0)),
        pl.BlockSpec((NB, 1), lambda i: (i, 0)),
        pl.BlockSpec((H, H), lambda i: (0, 0)),
        pl.BlockSpec((1, H), lambda i: (0, 0)),
        pl.BlockSpec((H, 3 * H), lambda i: (0, 0)),
        pl.BlockSpec((H, 3 * H), lambda i: (0, 0)),
        pl.BlockSpec((1, 3 * H), lambda i: (0, 0)),
        pl.BlockSpec((1, 3 * H), lambda i: (0, 0)),
        pl.BlockSpec((H, C), lambda i: (0, 0)),
        pl.BlockSpec((1, C), lambda i: (0, 0)),
    ],
    out_specs=pl.BlockSpec((NB, C), lambda i: (i, 0)),
    out_shape=jax.ShapeDtypeStruct((NP, C), F32),
)


def kernel(x, W1, b1, W2, b2, W_ih, W_hh, b_ih, b_hh, Wo, bo, edge_index):
    x2 = jnp.pad(x.reshape(N, K * F_IN), ((0, NP - N), (0, F1 - K * F_IN)))
    ones16 = jnp.ones((BATCH, 16), F32)
    zeros16 = jnp.zeros((STRIPE, 16), F32)
    zerosfc = jnp.zeros((STRIPE, FC), F32)
    src = edge_index[0]
    dst = edge_index[1]
    padi = jnp.full((EP - E,), N, jnp.int32)
    src_p = jnp.concatenate([src, padi])
    dst_p = jnp.concatenate([dst, padi])
    src1 = src_p.reshape(SCS * TILES, NB1, BATCH)
    dst1 = dst_p.reshape(SCS * TILES, NB1, BATCH)
    src2 = src_p.reshape(TILES, NB2, BATCH)
    dst2 = dst_p.reshape(TILES, NB2, BATCH)

    degp = _get_deg_kernel()(dst1, ones16, zeros16)
    u1, dinv = _prep(degp, x2)
    y1 = _get_scat_kernel(NCH1)(u1.reshape(NCH1 * NP, FC), src2, dst2,
                                zerosfc).reshape(NCH1, NP, FC)
    v = _mid(y1, u1, dinv, W1, b1.reshape(1, H))
    z = _get_scat_kernel(NCH)(v.reshape(NCH * NP, FC), src2, dst2,
                              zerosfc).reshape(NCH, NP, FC)
    outp = _gru(z, v, dinv, W2, b2.reshape(1, H),
                W_ih.T, W_hh.T, b_ih.reshape(1, 3 * H), b_hh.reshape(1, 3 * H),
                Wo, bo.reshape(1, C))
    return outp[:N]


# speedup vs baseline: 18.4027x; 2.0181x over previous
import functools

import jax
import jax.numpy as jnp
from jax import lax
from jax.experimental import pallas as pl
from jax.experimental.pallas import tpu as pltpu
from jax.experimental.pallas import tpu_sc as plsc

N = 50000
E = 800000
H = 64
F_IN = 5
K = 7
C = 5

NP = 50176
EP = 802816
NB = 512
GRID_N = NP // NB
TILES = 16
SCS = 2
BATCH = 128
NB1 = EP // (SCS * TILES * BATCH)
NB2 = EP // (TILES * BATCH)
STRIPE = NP // TILES
FC = 32
NCH1 = 2
NCH = (K * H) // FC
F1 = NCH1 * FC
F32 = jnp.float32


def _sc_mesh():
    return plsc.VectorSubcoreMesh(core_axis_name="c", subcore_axis_name="s")


@functools.cache
def _get_deg_kernel():
    return functools.partial(
        pl.kernel,
        out_type=jax.ShapeDtypeStruct((SCS, NP, 16), F32),
        mesh=_sc_mesh(),
        compiler_params=pltpu.CompilerParams(use_tc_tiling_on_sc=False),
        scratch_types=[
            pltpu.VMEM((BATCH,), jnp.int32),
            pltpu.VMEM((BATCH, 16), F32),
            pltpu.VMEM_SHARED((NP, 16), F32),
        ],
    )(_deg_body)


def _deg_body(dst3, ones16, zeros16, out, idx_v, ones_v, acc):
    cid = lax.axis_index("c")
    sid = lax.axis_index("s")
    w = cid * TILES + sid
    base = sid * STRIPE
    pltpu.sync_copy(ones16, ones_v)
    pltpu.sync_copy(zeros16, acc.at[pl.ds(base, STRIPE)])
    plsc.subcore_barrier()

    def body(b, carry):
        pltpu.sync_copy(dst3.at[w, b], idx_v)
        pltpu.sync_copy(ones_v, acc.at[idx_v], add=True)
        return carry

    lax.fori_loop(0, NB1, body, 0)
    plsc.subcore_barrier()
    pltpu.sync_copy(acc.at[pl.ds(base, STRIPE)], out.at[cid, pl.ds(base, STRIPE)])


GRP = 56
NGRP = NB2 // GRP


@functools.cache
def _get_scat_kernel(nch):
    return functools.partial(
        pl.kernel,
        out_type=jax.ShapeDtypeStruct((nch, NP, FC), F32),
        mesh=_sc_mesh(),
        compiler_params=pltpu.CompilerParams(use_tc_tiling_on_sc=False),
        scratch_types=[
            pltpu.VMEM((GRP, BATCH), jnp.int32),
            pltpu.VMEM((GRP, BATCH), jnp.int32),
            pltpu.VMEM((2, BATCH, FC), F32),
            pltpu.VMEM_SHARED((NP, FC), F32),
            pltpu.SemaphoreType.DMA,
            pltpu.SemaphoreType.DMA,
        ],
    )(functools.partial(_scat_body, nch))


def _scat_body(nch, table, src3, dst3, zerosfc, out, src_sl, dst_sl, ring, acc,
               gsem, ssem):
    cid = lax.axis_index("c")
    sid = lax.axis_index("s")
    base = sid * STRIPE
    pltpu.sync_copy(zerosfc, acc.at[pl.ds(base, STRIPE)])

    def chunk_body(cl, carry):
        ch = cid * (nch // SCS) + cl
        plsc.subcore_barrier()

        def fire_gather(m, slot):
            pltpu.async_copy(table.at[ch].at[src_sl.at[m]], ring.at[slot], gsem)

        def wait_gather(slot):
            pltpu.make_async_copy(table.at[ch].at[src_sl.at[0]],
                                  ring.at[slot], gsem).wait()

        def fire_scatter(m, slot):
            pltpu.async_copy(ring.at[slot], acc.at[dst_sl.at[m]], ssem,
                             add=True)

        def wait_scatter(slot):
            pltpu.make_async_copy(ring.at[slot], acc.at[dst_sl.at[0]],
                                  ssem).wait()

        def group_body(g, c1):
            pltpu.sync_copy(src3.at[sid, pl.ds(g * GRP, GRP)], src_sl)
            pltpu.sync_copy(dst3.at[sid, pl.ds(g * GRP, GRP)], dst_sl)
            fire_gather(0, 0)

            def body(m, c2):
                @pl.when(m >= 1)
                def _drain():
                    wait_scatter((m - 1) % 2)

                @pl.when(m + 1 < GRP)
                def _prefetch():
                    fire_gather(m + 1, (m + 1) % 2)

                wait_gather(m % 2)
                fire_scatter(m, m % 2)
                return c2

            lax.fori_loop(0, GRP, body, 0)
            wait_scatter((GRP - 1) % 2)
            return c1

        lax.fori_loop(0, NGRP, group_body, 0)
        plsc.subcore_barrier()
        pltpu.sync_copy(acc.at[pl.ds(base, STRIPE)],
                        out.at[ch, pl.ds(base, STRIPE)])
        pltpu.sync_copy(zerosfc, acc.at[pl.ds(base, STRIPE)])
        return carry

    lax.fori_loop(0, nch // SCS, chunk_body, 0)


def _prep_body(degp, x2, u1, dinv):
    dd = degp[...]
    d = dd[0, :, 0] + dd[1, :, 0] + 1.0
    di = 1.0 / jnp.sqrt(d)
    dinv[...] = di[:, None]
    u = x2[...] * di[:, None]
    u1[0] = u[:, :FC]
    u1[1] = u[:, FC:]


_prep = pl.pallas_call(
    _prep_body,
    grid=(GRID_N,),
    in_specs=[
        pl.BlockSpec((SCS, NB, 16), lambda i: (0, i, 0)),
        pl.BlockSpec((NB, F1), lambda i: (i, 0)),
    ],
    out_specs=[
        pl.BlockSpec((NCH1, NB, FC), lambda i: (0, i, 0)),
        pl.BlockSpec((NB, 1), lambda i: (i, 0)),
    ],
    out_shape=[
        jax.ShapeDtypeStruct((NCH1, NP, FC), F32),
        jax.ShapeDtypeStruct((NP, 1), F32),
    ],
)


def _mid_body(y1, u1, dinv, w1, b1, v):
    di = dinv[...]
    ys = jnp.concatenate([y1[0] + u1[0], y1[1] + u1[1]], axis=1) * di
    for k in range(K):
        xs = jnp.dot(ys[:, 5 * k:5 * k + 5], w1[...],
                     preferred_element_type=F32,
                     precision=lax.Precision.HIGHEST)
        xs = jnp.maximum(xs + b1[...], 0.0)
        vk = xs * di
        v[2 * k] = vk[:, :FC]
        v[2 * k + 1] = vk[:, FC:]


_mid = pl.pallas_call(
    _mid_body,
    grid=(GRID_N,),
    in_specs=[
        pl.BlockSpec((NCH1, NB, FC), lambda i: (0, i, 0)),
        pl.BlockSpec((NCH1, NB, FC), lambda i: (0, i, 0)),
        pl.BlockSpec((NB, 1), lambda i: (i, 0)),
        pl.BlockSpec((F_IN, H), lambda i: (0, 0)),
        pl.BlockSpec((1, H), lambda i: (0, 0)),
    ],
    out_specs=pl.BlockSpec((NCH, NB, FC), lambda i: (0, i, 0)),
    out_shape=jax.ShapeDtypeStruct((NCH, NP, FC), F32),
)


def _gru_body(z, vv, dinv, w2, b2, wiht, whht, bih, bhh, wo, bo, out):
    di = dinv[...]
    h = jnp.zeros((NB, H), F32)
    for k in range(K):
        za = (z[2 * k] + vv[2 * k]) * di
        zb = (z[2 * k + 1] + vv[2 * k + 1]) * di
        z2 = jnp.concatenate([za, zb], axis=1)
        xs2 = jnp.dot(z2, w2[...], preferred_element_type=F32,
                      precision=lax.Precision.HIGHEST)
        xs2 = jnp.maximum(xs2 + b2[...], 0.0)
        gi = jnp.dot(xs2, wiht[...], preferred_element_type=F32,
                     precision=lax.Precision.HIGHEST) + bih[...]
        gh = jnp.dot(h, whht[...], preferred_element_type=F32,
                     precision=lax.Precision.HIGHEST) + bhh[...]
        r = jax.nn.sigmoid(gi[:, :H] + gh[:, :H])
        zz = jax.nn.sigmoid(gi[:, H:2 * H] + gh[:, H:2 * H])
        n = jnp.tanh(gi[:, 2 * H:] + r * gh[:, 2 * H:])
        h = (1.0 - zz) * n + zz * h
    logits = jnp.dot(h, wo[...], preferred_element_type=F32,
                     precision=lax.Precision.HIGHEST) + bo[...]
    m = jnp.max(logits, axis=1, keepdims=True)
    p = jnp.exp(logits - m)
    out[...] = p / jnp.sum(p, axis=1, keepdims=True)


_gru = pl.pallas_call(
    _gru_body,
    grid=(GRID_N,),
    in_specs=[
        pl.BlockSpec((NCH, NB, FC), lambda i: (0, i, 0)),
        pl.BlockSpec((NCH, NB, FC), lambda i: (0, i, 0)),
        pl.BlockSpec((NB, 1), lambda i: (i, 0)),
        pl.BlockSpec((H, H), lambda i: (0, 0)),
        pl.BlockSpec((1, H), lambda i: (0, 0)),
        pl.BlockSpec((H, 3 * H), lambda i: (0, 0)),
        pl.BlockSpec((H, 3 * H), lambda i: (0, 0)),
        pl.BlockSpec((1, 3 * H), lambda i: (0, 0)),
        pl.BlockSpec((1, 3 * H), lambda i: (0, 0)),
        pl.BlockSpec((H, C), lambda i: (0, 0)),
        pl.BlockSpec((1, C), lambda i: (0, 0)),
    ],
    out_specs=pl.BlockSpec((NB, C), lambda i: (i, 0)),
    out_shape=jax.ShapeDtypeStruct((NP, C), F32),
)


def kernel(x, W1, b1, W2, b2, W_ih, W_hh, b_ih, b_hh, Wo, bo, edge_index):
    x2 = jnp.pad(x.reshape(N, K * F_IN), ((0, NP - N), (0, F1 - K * F_IN)))
    ones16 = jnp.ones((BATCH, 16), F32)
    zeros16 = jnp.zeros((STRIPE, 16), F32)
    zerosfc = jnp.zeros((STRIPE, FC), F32)
    src = edge_index[0]
    dst = edge_index[1]
    padi = jnp.full((EP - E,), N, jnp.int32)
    src_p = jnp.concatenate([src, padi])
    dst_p = jnp.concatenate([dst, padi])
    src1 = src_p.reshape(SCS * TILES, NB1, BATCH)
    dst1 = dst_p.reshape(SCS * TILES, NB1, BATCH)
    src2 = src_p.reshape(TILES, NB2, BATCH)
    dst2 = dst_p.reshape(TILES, NB2, BATCH)

    degp = _get_deg_kernel()(dst1, ones16, zeros16)
    u1, dinv = _prep(degp, x2)
    y1 = _get_scat_kernel(NCH1)(u1, src2, dst2, zerosfc)
    v = _mid(y1, u1, dinv, W1, b1.reshape(1, H))
    z = _get_scat_kernel(NCH)(v, src2, dst2, zerosfc)
    outp = _gru(z, v, dinv, W2, b2.reshape(1, H),
                W_ih.T, W_hh.T, b_ih.reshape(1, 3 * H), b_hh.reshape(1, 3 * H),
                Wo, bo.reshape(1, C))
    return outp[:N]
